# Initial kernel scaffold; baseline (speedup 1.0000x reference)
import jax
import jax.numpy as jnp
from jax.experimental import pallas as pl


def kernel(x, We1, be1, We2, be2, Wn1, bn1, Wn2, bn2, gamma, beta, edge_index):
    raise NotImplementedError("write your pallas kernel here")


# speedup vs baseline: 3.3690x; 3.3690x over previous
import functools

import jax
import jax.numpy as jnp
from jax import lax
from jax.experimental import pallas as pl
from jax.experimental.pallas import tpu as pltpu
from jax.experimental.pallas import tpu_sc as plsc

_N = 10000
_H = 256
_E = 160000
_HH = 128
_LANES = 16
_NC = 2
_NT = 16
_EPT = _E // _NT
_K = 80
_NCH = _EPT // _K
_NZ = 624
_NREM = _N - _NT * _NZ
_R = 1000
_G = _N // _R


def _pre_body(x_ref, w_ref, b_ref, a_ref, bb_ref):
    xb = x_ref[...]
    w = w_ref[...]
    a_ref[...] = (
        jnp.dot(xb, w[:_H, :], preferred_element_type=jnp.float32) + b_ref[...]
    )
    bb_ref[...] = jnp.dot(xb, w[_H:, :], preferred_element_type=jnp.float32)


def _pre_projections(x, We1_i, be1_row):
    return pl.pallas_call(
        _pre_body,
        grid=(_G, _NC),
        in_specs=[
            pl.BlockSpec((_R, _H), lambda i, h: (i, 0)),
            pl.BlockSpec((2 * _H, _HH), lambda i, h: (0, h)),
            pl.BlockSpec((1, _HH), lambda i, h: (0, h)),
        ],
        out_specs=[
            pl.BlockSpec((_R, _HH), lambda i, h: (h * _G + i, 0)),
            pl.BlockSpec((_R, _HH), lambda i, h: (h * _G + i, 0)),
        ],
        out_shape=[jax.ShapeDtypeStruct((_NC * _N, _HH), jnp.float32)] * 2,
    )(x, We1_i, be1_row)


def _post_body(s0_ref, s1_ref, x_ref, cnt_ref, we2_ref, wn1_ref, wn2_ref,
               be2_ref, bn1_ref, bn2_ref, g_ref, bt_ref, out_ref):
    s = jnp.concatenate([s0_ref[...], s1_ref[...]], axis=1)
    agg = (
        jnp.dot(s, we2_ref[...], preferred_element_type=jnp.float32)
        + cnt_ref[...] * be2_ref[...]
    )
    xb = x_ref[...]
    wn1 = wn1_ref[...]
    h1 = jnp.maximum(
        jnp.dot(xb, wn1[:_H, :], preferred_element_type=jnp.float32)
        + jnp.dot(agg, wn1[_H:, :], preferred_element_type=jnp.float32)
        + bn1_ref[...],
        0.0,
    )
    u = jnp.dot(h1, wn2_ref[...], preferred_element_type=jnp.float32) + bn2_ref[...]
    y = u + xb
    mu = jnp.mean(y, axis=1, keepdims=True)
    var = jnp.mean((y - mu) ** 2, axis=1, keepdims=True)
    out_ref[...] = (y - mu) * lax.rsqrt(var + 1e-5) * g_ref[...] + bt_ref[...]


def _post_update(s0, s1, x, counts, We2_i, Wn1_i, Wn2_i, be2_row, bn1_row,
                 bn2_row, g_row, bt_row):
    full = lambda shape: pl.BlockSpec(shape, lambda i: (0, 0))
    return pl.pallas_call(
        _post_body,
        grid=(_G,),
        in_specs=[
            pl.BlockSpec((_R, _HH), lambda i: (i, 0)),
            pl.BlockSpec((_R, _HH), lambda i: (i, 0)),
            pl.BlockSpec((_R, _H), lambda i: (i, 0)),
            pl.BlockSpec((_R, 1), lambda i: (i, 0)),
            full((_H, _H)),
            full((2 * _H, _H)),
            full((_H, _H)),
            full((1, _H)),
            full((1, _H)),
            full((1, _H)),
            full((1, _H)),
            full((1, _H)),
        ],
        out_specs=pl.BlockSpec((_R, _H), lambda i: (i, 0)),
        out_shape=jax.ShapeDtypeStruct((_N, _H), jnp.float32),
    )(s0, s1, x, counts, We2_i, Wn1_i, Wn2_i, be2_row, bn1_row, bn2_row,
      g_row, bt_row)


def _make_sc_edge_kernel():
    mesh = plsc.VectorSubcoreMesh(core_axis_name="c", subcore_axis_name="s")

    @functools.partial(
        pl.kernel,
        mesh=mesh,
        out_type=jax.ShapeDtypeStruct((_NC, _N, _HH), jnp.float32),
        scratch_types=[
            pltpu.VMEM((_NCH, _K), jnp.int32),
            pltpu.VMEM((_K,), jnp.int32),
            pltpu.VMEM((_K,), jnp.int32),
            pltpu.VMEM((_NCH, _K), jnp.int32),
            pltpu.VMEM((_K, _HH), jnp.float32),
            pltpu.VMEM_SHARED((_N, _HH), jnp.float32),
        ],
    )
    def sc_edge(a_h, b_h, src_h, dst_h, out_h,
                srcv, srcg, dstg, dstsv, ra, s_sh):
        c = lax.axis_index("c")
        t = lax.axis_index("s")

        def zrow(r, carry):
            for u in range(_HH // _LANES):
                ra[r, pl.ds(u * _LANES, _LANES)] = jnp.zeros(
                    (_LANES,), jnp.float32)
            return carry
        lax.fori_loop(0, _K, zrow, 0)
        for q in range(_NZ // _K):
            pltpu.sync_copy(ra, s_sh.at[pl.ds(t * _NZ + q * _K, _K)])
        zrem = _NZ - (_NZ // _K) * _K
        if zrem:
            pltpu.sync_copy(
                ra.at[pl.ds(0, zrem)],
                s_sh.at[pl.ds(t * _NZ + (_NZ // _K) * _K, zrem)])

        @pl.when(t == _NT - 1)
        def _zero_rem():
            pltpu.sync_copy(ra.at[pl.ds(0, _NREM)],
                            s_sh.at[pl.ds(_NT * _NZ, _NREM)])
        plsc.subcore_barrier()

        pltpu.sync_copy(src_h.at[t], srcv)
        pltpu.sync_copy(dst_h.at[t], dstsv)
        cn = jnp.broadcast_to((c * _N).astype(jnp.int32), (_LANES,))

        def chunk(j, carry):
            for u in range(_K // _LANES):
                sl = pl.ds(u * _LANES, _LANES)
                srcg[sl] = srcv[j, sl] + cn
                dstg[sl] = dstsv[j, sl] + cn

            pltpu.sync_copy(a_h.at[srcg], ra)
            pltpu.sync_copy(b_h.at[dstg], ra, add=True)

            def relu_row(r, inner):
                for u in range(_HH // _LANES):
                    sl = pl.ds(u * _LANES, _LANES)
                    ra[r, sl] = jnp.maximum(ra[r, sl], 0.0)
                return inner
            lax.fori_loop(0, _K, relu_row, 0)

            pltpu.sync_copy(ra, s_sh.at[dstsv.at[j]], add=True)
            return carry
        lax.fori_loop(0, _NCH, chunk, 0)

        plsc.subcore_barrier()
        pltpu.sync_copy(s_sh.at[pl.ds(t * _NZ, _NZ)],
                        out_h.at[c, pl.ds(t * _NZ, _NZ)])

        @pl.when(t == _NT - 1)
        def _copy_rem():
            pltpu.sync_copy(s_sh.at[pl.ds(_NT * _NZ, _NREM)],
                            out_h.at[c, pl.ds(_NT * _NZ, _NREM)])

    return sc_edge


_sc_edge = _make_sc_edge_kernel()


def kernel(x, We1, be1, We2, be2, Wn1, bn1, Wn2, bn2, gamma, beta, edge_index):
    L = We1.shape[0]
    assert x.shape == (_N, _H) and edge_index.shape == (2, _E)

    src = edge_index[0]
    dst = edge_index[1]

    src_r = src.reshape(_NT, _NCH, _K)
    dst_r = dst.reshape(_NT, _NCH, _K)

    counts = jnp.bincount(dst, length=_N).astype(jnp.float32).reshape(_N, 1)

    row = lambda v: v.reshape(1, _H)
    for i in range(L):
        a_fl, b_fl = _pre_projections(x, We1[i], row(be1[i]))
        s = _sc_edge(a_fl, b_fl, src_r, dst_r)
        x = _post_update(s[0], s[1], x, counts, We2[i], Wn1[i], Wn2[i],
                         row(be2[i]), row(bn1[i]), row(bn2[i]),
                         row(gamma[i]), row(beta[i]))
    return x
